# baseline (speedup 1.0000x reference)
import jax
import jax.numpy as jnp
from jax import lax
from jax.experimental import pallas as pl
from jax.experimental.pallas import tpu as pltpu
from jax.experimental.pallas import tpu_sc as plsc

_N = 10000
_E = 320000
_D = 128
_EPS = 1e-5

_NC = 2
_NS = 16
_NW = _NC * _NS
_CH = 112
_NCHUNK = 90
_EPW = _NCHUNK * _CH
_RPT = 624
_RTAIL = _N - _NS * _RPT
_NPAD = 8


def _matmul_body(x_ref, w_ref, z_ref):
    z_ref[...] = lax.dot_general(
        x_ref[...].astype(jnp.bfloat16), w_ref[...].astype(jnp.bfloat16),
        dimension_numbers=(((1,), (1,)), ((), ())),
        preferred_element_type=jnp.float32,
    )


def _project(x_padded, W):
    return pl.pallas_call(
        _matmul_body,
        out_shape=jax.ShapeDtypeStruct((_N + _NPAD, _D), jnp.float32),
    )(x_padded, W)


def _sc_body(z_hbm, src_hbm, dst_hbm, out_hbm,
             srcs, dsts, rows0, rows1, hpart,
             gsem0, gsem1, ssem0, ssem1):
    cid = lax.axis_index("core")
    sid = lax.axis_index("subcore")
    wid = cid * _NS + sid

    pltpu.sync_copy(src_hbm.at[wid], srcs)
    pltpu.sync_copy(dst_hbm.at[wid], dsts)

    @pl.loop(0, _CH)
    def _zero_rows(i):
        @pl.loop(0, _D, step=16)
        def _zero_lane(j):
            rows0[i, pl.ds(j, 16)] = jnp.zeros((16,), jnp.float32)

    row0 = sid * _RPT
    @pl.loop(0, _RPT // _CH)
    def _zero_hpart(k):
        pltpu.sync_copy(rows0, hpart.at[pl.ds(row0 + k * _CH, _CH)])
    _tail = _RPT % _CH
    pltpu.sync_copy(rows0.at[pl.ds(0, _tail)],
                    hpart.at[pl.ds(row0 + _RPT - _tail, _tail)])

    @pl.when(sid == _NS - 1)
    def _zero_last():
        pltpu.sync_copy(rows0.at[pl.ds(0, _RTAIL)],
                        hpart.at[pl.ds(_NS * _RPT, _RTAIL)])

    def _src_slice(j):
        return srcs.at[pl.ds(j * _CH, _CH)]

    pltpu.async_copy(z_hbm.at[_src_slice(0)], rows0, gsem0)
    pltpu.async_copy(z_hbm.at[_src_slice(1)], rows1, gsem1)

    plsc.subcore_barrier()

    @pl.loop(0, _NCHUNK // 2)
    def _pair(k):
        j = 2 * k
        pltpu.make_async_copy(z_hbm.at[_src_slice(j)], rows0, gsem0).wait()
        pltpu.async_copy(rows0, hpart.at[dsts.at[j]], ssem0, add=True)
        pltpu.make_async_copy(z_hbm.at[_src_slice(j + 1)], rows1, gsem1).wait()
        pltpu.async_copy(rows1, hpart.at[dsts.at[j + 1]], ssem1, add=True)
        pltpu.make_async_copy(rows0, hpart.at[dsts.at[j]], ssem0).wait()
        @pl.when(j + 2 < _NCHUNK)
        def _next0():
            pltpu.async_copy(z_hbm.at[_src_slice(j + 2)], rows0, gsem0)
        pltpu.make_async_copy(rows1, hpart.at[dsts.at[j + 1]], ssem1).wait()
        @pl.when(j + 3 < _NCHUNK)
        def _next1():
            pltpu.async_copy(z_hbm.at[_src_slice(j + 3)], rows1, gsem1)

    plsc.subcore_barrier()

    pltpu.sync_copy(hpart.at[pl.ds(row0, _RPT)],
                    out_hbm.at[cid, pl.ds(row0, _RPT)])

    @pl.when(sid == _NS - 1)
    def _write_last():
        pltpu.sync_copy(hpart.at[pl.ds(_NS * _RPT, _RTAIL)],
                        out_hbm.at[cid, pl.ds(_NS * _RPT, _RTAIL)])


def _sc_aggregate(z, src, dst):
    mesh = plsc.VectorSubcoreMesh(core_axis_name="core",
                                  subcore_axis_name="subcore")
    f = pl.kernel(
        _sc_body,
        out_type=jax.ShapeDtypeStruct((_NC, _N, _D), jnp.float32),
        mesh=mesh,
        scratch_types=[
            pltpu.VMEM((_EPW,), jnp.int32),
            pltpu.VMEM((_NCHUNK, _CH), jnp.int32),
            pltpu.VMEM((_CH, _D), jnp.float32),
            pltpu.VMEM((_CH, _D), jnp.float32),
            pltpu.VMEM_SHARED((_N, _D), jnp.float32),
            pltpu.SemaphoreType.DMA,
            pltpu.SemaphoreType.DMA,
            pltpu.SemaphoreType.DMA,
            pltpu.SemaphoreType.DMA,
        ],
    )
    return f(z, src, dst)


def _bn_body(p_ref, g_ref, b_ref, o_ref):
    h = p_ref[0] + p_ref[1]
    mean = jnp.mean(h, axis=0, keepdims=True)
    c = h - mean
    var = jnp.mean(c * c, axis=0, keepdims=True)
    hn = c * lax.rsqrt(var + _EPS) * g_ref[...][None, :] + b_ref[...][None, :]
    o_ref[...] = jnp.where(hn > 0, hn, jnp.exp(jnp.minimum(hn, 0.0)) - 1.0)


def _bn_elu(parts, gamma, beta):
    return pl.pallas_call(
        _bn_body,
        out_shape=jax.ShapeDtypeStruct((_N, _D), jnp.float32),
    )(parts, gamma, beta)


def kernel(x, edge_index, W, gamma, beta):
    x_padded = jnp.pad(x, ((0, _NPAD), (0, 0)))
    z = _project(x_padded, W)
    pad_e = _EPW - _E // _NW
    src = jnp.concatenate(
        [edge_index[0].reshape(_NW, _E // _NW),
         jnp.full((_NW, pad_e), _N, jnp.int32)], axis=1)
    dst = jnp.concatenate(
        [edge_index[1].reshape(_NW, _E // _NW),
         jnp.zeros((_NW, pad_e), jnp.int32)], axis=1)
    parts = _sc_aggregate(z, src, dst.reshape(_NW, _NCHUNK, _CH))
    return _bn_elu(parts, gamma, beta)


# speedup vs baseline: 1.4565x; 1.4565x over previous
import jax
import jax.numpy as jnp
from jax import lax
from jax.experimental import pallas as pl
from jax.experimental.pallas import tpu as pltpu
from jax.experimental.pallas import tpu_sc as plsc

_N = 10000
_E = 320000
_D = 128
_EPS = 1e-5

_NC = 2
_NS = 16
_NW = _NC * _NS
_CH = 112
_NCHUNK = 90
_EPW = _NCHUNK * _CH
_RPT = 624
_RTAIL = _N - _NS * _RPT
_NPAD = 80


def _matmul_body(x_ref, w_ref, z_ref):
    z_ref[...] = lax.dot_general(
        x_ref[...].astype(jnp.bfloat16), w_ref[...].astype(jnp.bfloat16),
        dimension_numbers=(((1,), (1,)), ((), ())),
        preferred_element_type=jnp.float32,
    )


def _project(x_padded, W):
    return pl.pallas_call(
        _matmul_body,
        out_shape=jax.ShapeDtypeStruct((_N + _NPAD, _D), jnp.float32),
    )(x_padded, W)


def _sc_body(z_hbm, src_hbm, dst_hbm, out_hbm,
             srcs, dsts, rows0, rows1, hpart,
             gsem0, gsem1, ssem0, ssem1):
    cid = lax.axis_index("core")
    sid = lax.axis_index("subcore")
    wid = cid * _NS + sid

    pltpu.sync_copy(src_hbm.at[wid], srcs)
    pltpu.sync_copy(dst_hbm.at[wid], dsts)

    @pl.loop(0, _CH)
    def _zero_rows(i):
        @pl.loop(0, _D, step=16)
        def _zero_lane(j):
            rows0[i, pl.ds(j, 16)] = jnp.zeros((16,), jnp.float32)

    row0 = sid * _RPT
    @pl.loop(0, _RPT // _CH)
    def _zero_hpart(k):
        pltpu.sync_copy(rows0, hpart.at[pl.ds(row0 + k * _CH, _CH)])
    _tail = _RPT % _CH
    pltpu.sync_copy(rows0.at[pl.ds(0, _tail)],
                    hpart.at[pl.ds(row0 + _RPT - _tail, _tail)])

    @pl.when(sid == _NS - 1)
    def _zero_last():
        pltpu.sync_copy(rows0.at[pl.ds(0, _RTAIL)],
                        hpart.at[pl.ds(_NS * _RPT, _RTAIL)])

    def _src_slice(j):
        return srcs.at[pl.ds(j * _CH, _CH)]

    pltpu.async_copy(z_hbm.at[_src_slice(0)], rows0, gsem0)
    pltpu.async_copy(z_hbm.at[_src_slice(1)], rows1, gsem1)

    plsc.subcore_barrier()

    @pl.loop(0, _NCHUNK // 2)
    def _pair(k):
        j = 2 * k
        pltpu.make_async_copy(z_hbm.at[_src_slice(j)], rows0, gsem0).wait()
        pltpu.async_copy(rows0, hpart.at[dsts.at[j]], ssem0, add=True)
        pltpu.make_async_copy(z_hbm.at[_src_slice(j + 1)], rows1, gsem1).wait()
        pltpu.async_copy(rows1, hpart.at[dsts.at[j + 1]], ssem1, add=True)
        pltpu.make_async_copy(rows0, hpart.at[dsts.at[j]], ssem0).wait()
        @pl.when(j + 2 < _NCHUNK)
        def _next0():
            pltpu.async_copy(z_hbm.at[_src_slice(j + 2)], rows0, gsem0)
        pltpu.make_async_copy(rows1, hpart.at[dsts.at[j + 1]], ssem1).wait()
        @pl.when(j + 3 < _NCHUNK)
        def _next1():
            pltpu.async_copy(z_hbm.at[_src_slice(j + 3)], rows1, gsem1)

    plsc.subcore_barrier()

    pltpu.sync_copy(hpart.at[pl.ds(row0, _RPT)],
                    out_hbm.at[cid, pl.ds(row0, _RPT)])

    @pl.when(sid == _NS - 1)
    def _write_last():
        pltpu.sync_copy(hpart.at[pl.ds(_NS * _RPT, _RTAIL)],
                        out_hbm.at[cid, pl.ds(_NS * _RPT, _RTAIL)])


def _sc_aggregate(z, src, dst):
    mesh = plsc.VectorSubcoreMesh(core_axis_name="core",
                                  subcore_axis_name="subcore")
    f = pl.kernel(
        _sc_body,
        out_type=jax.ShapeDtypeStruct((_NC, _N, _D), jnp.float32),
        mesh=mesh,
        scratch_types=[
            pltpu.VMEM((_EPW,), jnp.int32),
            pltpu.VMEM((_NCHUNK, _CH), jnp.int32),
            pltpu.VMEM((_CH, _D), jnp.float32),
            pltpu.VMEM((_CH, _D), jnp.float32),
            pltpu.VMEM_SHARED((_N, _D), jnp.float32),
            pltpu.SemaphoreType.DMA,
            pltpu.SemaphoreType.DMA,
            pltpu.SemaphoreType.DMA,
            pltpu.SemaphoreType.DMA,
        ],
    )
    return f(z, src, dst)


def _bn_body(p_ref, g_ref, b_ref, o_ref):
    h = p_ref[0] + p_ref[1]
    mean = jnp.mean(h, axis=0, keepdims=True)
    c = h - mean
    var = jnp.mean(c * c, axis=0, keepdims=True)
    hn = c * lax.rsqrt(var + _EPS) * g_ref[...][None, :] + b_ref[...][None, :]
    o_ref[...] = jnp.where(hn > 0, hn, jnp.exp(jnp.minimum(hn, 0.0)) - 1.0)


def _bn_elu(parts, gamma, beta):
    return pl.pallas_call(
        _bn_body,
        out_shape=jax.ShapeDtypeStruct((_N, _D), jnp.float32),
    )(parts, gamma, beta)


def kernel(x, edge_index, W, gamma, beta):
    x_padded = jnp.pad(x, ((0, _NPAD), (0, 0)))
    z = _project(x_padded, W)
    pad_e = _EPW - _E // _NW
    pad_src = jnp.broadcast_to(_N + jnp.arange(pad_e, dtype=jnp.int32) % _NPAD,
                               (_NW, pad_e))
    pad_dst = jnp.broadcast_to(jnp.arange(pad_e, dtype=jnp.int32) % _N,
                               (_NW, pad_e))
    src = jnp.concatenate(
        [edge_index[0].reshape(_NW, _E // _NW), pad_src], axis=1)
    dst = jnp.concatenate(
        [edge_index[1].reshape(_NW, _E // _NW), pad_dst], axis=1)
    parts = _sc_aggregate(z, src, dst.reshape(_NW, _NCHUNK, _CH))
    return _bn_elu(parts, gamma, beta)
